# baseline (speedup 1.0000x reference)
import functools

import jax
import jax.numpy as jnp
from jax import lax
from jax.experimental import pallas as pl
from jax.experimental.pallas import tpu as pltpu
from jax.experimental.pallas import tpu_sc as plsc

VOCAB_PAD = 1024
GCH = 512
LANES = 16


def _table_body(scale_ref, w_ref, out_ref):
    w = w_ref[...]
    nsq = jnp.sum(w * w, axis=1, keepdims=True)
    inv = 1.0 / jnp.maximum(jnp.sqrt(nsq), 1e-8)
    nrm = w * inv
    g = lax.dot_general(nrm, nrm, (((1,), (1,)), ((), ())),
                        preferred_element_type=jnp.float32,
                        precision=lax.Precision.HIGHEST)
    out_ref[...] = g * scale_ref[0]


def _build_table(w_pad, scale):
    return pl.pallas_call(
        _table_body,
        out_shape=jax.ShapeDtypeStruct((VOCAB_PAD, VOCAB_PAD), jnp.float32),
        in_specs=[
            pl.BlockSpec(memory_space=pltpu.SMEM),
            pl.BlockSpec(memory_space=pltpu.VMEM),
        ],
        out_specs=pl.BlockSpec(memory_space=pltpu.VMEM),
    )(scale, w_pad)


@functools.lru_cache(maxsize=None)
def _make_gather(num_pos, num_neg, neg_rate):
    info = plsc.get_sparse_core_info()
    nc, ns = info.num_cores, info.num_subcores
    nw = nc * ns
    p_cnt = num_pos // nw
    assert p_cnt * nw == num_pos and num_neg == num_pos * neg_rate
    assert p_cnt % GCH == 0 and p_cnt % LANES == 0
    nph = 1 + neg_rate
    mesh = plsc.VectorSubcoreMesh(core_axis_name="c", subcore_axis_name="s")

    @functools.partial(
        pl.kernel, mesh=mesh,
        out_type=jax.ShapeDtypeStruct((nw, LANES), jnp.float32),
        scratch_types=[
            pltpu.VMEM((p_cnt,), jnp.int32),
            pltpu.VMEM((p_cnt,), jnp.int32),
            pltpu.VMEM((neg_rate * p_cnt,), jnp.int32),
            pltpu.VMEM((p_cnt,), jnp.int32),
            pltpu.VMEM((p_cnt,), jnp.int32),
            pltpu.VMEM((p_cnt,), jnp.float32),
            pltpu.VMEM((p_cnt,), jnp.float32),
            pltpu.VMEM((LANES,), jnp.float32),
            pltpu.SemaphoreType.DMA,
            pltpu.SemaphoreType.DMA,
            pltpu.SemaphoreType.DMA,
            pltpu.SemaphoreType.DMA,
        ],
    )
    def gather_kernel(table_hbm, cen_hbm, pos_hbm, negt_hbm, out_hbm,
                      cb, pb, nb, ix_a, ix_b, vl_a, vl_b, accb,
                      sem_cp, sem_ng, sem_a, sem_b):
        wid = lax.axis_index("s") * nc + lax.axis_index("c")
        base = wid * p_cnt
        cp_c = pltpu.async_copy(cen_hbm.at[pl.ds(base, p_cnt)], cb, sem_cp)
        cp_p = pltpu.async_copy(pos_hbm.at[pl.ds(base, p_cnt)], pb, sem_cp)
        cp_n = [
            pltpu.async_copy(
                negt_hbm.at[pl.ds(rr * num_pos + base, p_cnt)],
                nb.at[pl.ds(rr * p_cnt, p_cnt)], sem_ng)
            for rr in range(neg_rate)
        ]
        cp_c.wait()
        cp_p.wait()

        def compute_idx(oth_ref, oth_off, ix_ref):
            def body(vi, carry):
                st = vi * LANES
                ix_ref[pl.ds(st, LANES)] = (
                    cb[pl.ds(st, LANES)] * VOCAB_PAD
                    + oth_ref[pl.ds(oth_off + st, LANES)])
                return carry
            lax.fori_loop(0, p_cnt // LANES, body, 0)

        def fire(ix_ref, vl_ref, sem):
            return [
                pltpu.async_copy(
                    table_hbm.at[ix_ref.at[pl.ds(j * GCH, GCH)]],
                    vl_ref.at[pl.ds(j * GCH, GCH)], sem)
                for j in range(p_cnt // GCH)
            ]

        def acc_into(vl_ref, acc):
            def body(vi, a):
                return a + vl_ref[pl.ds(vi * LANES, LANES)]
            return lax.fori_loop(0, p_cnt // LANES, body, acc)

        ix = [ix_a, ix_b]
        vl = [vl_a, vl_b]
        sems = [sem_a, sem_b]

        acc_p = jnp.zeros((LANES,), jnp.float32)
        acc_n = jnp.zeros((LANES,), jnp.float32)

        compute_idx(pb, 0, ix[0])
        inflight = fire(ix[0], vl[0], sems[0])
        for p in range(1, nph):
            if p == 1:
                for cp in cp_n:
                    cp.wait()
            compute_idx(nb, (p - 1) * p_cnt, ix[p % 2])
            nxt = fire(ix[p % 2], vl[p % 2], sems[p % 2])
            for cp in inflight:
                cp.wait()
            if p - 1 == 0:
                acc_p = acc_into(vl[0], acc_p)
            else:
                acc_n = acc_into(vl[(p - 1) % 2], acc_n)
            inflight = nxt
        for cp in inflight:
            cp.wait()
        acc_n = acc_into(vl[(nph - 1) % 2], acc_n)

        accb[...] = acc_n - float(neg_rate) * acc_p
        pltpu.sync_copy(accb, out_hbm.at[wid])

    return gather_kernel


def kernel(center, pos_word, neg_word, w2v):
    b, s = center.shape
    r = neg_word.shape[1] // s
    v = w2v.shape[0]
    scale = jnp.full((1,), 1.0 / (s * r * b * 2.0), dtype=jnp.float32)
    w_pad = jnp.pad(w2v, ((0, VOCAB_PAD - v), (0, 0)))
    table = _build_table(w_pad, scale).reshape(-1)
    cen_f = center.reshape(-1)
    pos_f = pos_word.reshape(-1)
    negt_f = jnp.swapaxes(neg_word.reshape(b, r, s), 0, 1).reshape(-1)
    gather = _make_gather(cen_f.size, negt_f.size, r)
    partials = gather(table, cen_f, pos_f, negt_f)
    return 1.0 + jnp.sum(partials)


# speedup vs baseline: 1.0126x; 1.0126x over previous
import functools

import jax
import jax.numpy as jnp
from jax import lax
from jax.experimental import pallas as pl
from jax.experimental.pallas import tpu as pltpu
from jax.experimental.pallas import tpu_sc as plsc

VOCAB_PAD = 1024
GCH = 128
LANES = 16


def _table_body(scale_ref, w_ref, out_ref):
    w = w_ref[...]
    nsq = jnp.sum(w * w, axis=1, keepdims=True)
    inv = 1.0 / jnp.maximum(jnp.sqrt(nsq), 1e-8)
    nrm = w * inv
    g = lax.dot_general(nrm, nrm, (((1,), (1,)), ((), ())),
                        preferred_element_type=jnp.float32,
                        precision=lax.Precision.HIGHEST)
    out_ref[...] = g * scale_ref[0]


def _build_table(w_pad, scale):
    return pl.pallas_call(
        _table_body,
        out_shape=jax.ShapeDtypeStruct((VOCAB_PAD, VOCAB_PAD), jnp.float32),
        in_specs=[
            pl.BlockSpec(memory_space=pltpu.SMEM),
            pl.BlockSpec(memory_space=pltpu.VMEM),
        ],
        out_specs=pl.BlockSpec(memory_space=pltpu.VMEM),
    )(scale, w_pad)


@functools.lru_cache(maxsize=None)
def _make_gather(num_pos, num_neg, neg_rate):
    info = plsc.get_sparse_core_info()
    nc, ns = info.num_cores, info.num_subcores
    nw = nc * ns
    p_cnt = num_pos // nw
    assert p_cnt * nw == num_pos and num_neg == num_pos * neg_rate
    assert p_cnt % GCH == 0 and p_cnt % LANES == 0
    nph = 1 + neg_rate
    mesh = plsc.VectorSubcoreMesh(core_axis_name="c", subcore_axis_name="s")

    @functools.partial(
        pl.kernel, mesh=mesh,
        out_type=jax.ShapeDtypeStruct((nw, LANES), jnp.float32),
        scratch_types=[
            pltpu.VMEM((p_cnt,), jnp.int32),
            pltpu.VMEM((p_cnt,), jnp.int32),
            pltpu.VMEM((neg_rate * p_cnt,), jnp.int32),
            pltpu.VMEM((p_cnt,), jnp.int32),
            pltpu.VMEM((p_cnt,), jnp.int32),
            pltpu.VMEM((p_cnt,), jnp.float32),
            pltpu.VMEM((p_cnt,), jnp.float32),
            pltpu.VMEM((LANES,), jnp.float32),
            pltpu.SemaphoreType.DMA,
            pltpu.SemaphoreType.DMA,
            pltpu.SemaphoreType.DMA,
            pltpu.SemaphoreType.DMA,
        ],
    )
    def gather_kernel(table_hbm, cen_hbm, pos_hbm, negt_hbm, out_hbm,
                      cb, pb, nb, ix_a, ix_b, vl_a, vl_b, accb,
                      sem_cp, sem_ng, sem_a, sem_b):
        wid = lax.axis_index("s") * nc + lax.axis_index("c")
        base = wid * p_cnt
        cp_c = pltpu.async_copy(cen_hbm.at[pl.ds(base, p_cnt)], cb, sem_cp)
        cp_p = pltpu.async_copy(pos_hbm.at[pl.ds(base, p_cnt)], pb, sem_cp)
        cp_n = [
            pltpu.async_copy(
                negt_hbm.at[pl.ds(rr * num_pos + base, p_cnt)],
                nb.at[pl.ds(rr * p_cnt, p_cnt)], sem_ng)
            for rr in range(neg_rate)
        ]
        cp_c.wait()
        cp_p.wait()

        def compute_idx(oth_ref, oth_off, ix_ref):
            def body(vi, carry):
                st = vi * LANES
                ix_ref[pl.ds(st, LANES)] = (
                    cb[pl.ds(st, LANES)] * VOCAB_PAD
                    + oth_ref[pl.ds(oth_off + st, LANES)])
                return carry
            lax.fori_loop(0, p_cnt // LANES, body, 0)

        def fire(ix_ref, vl_ref, sem):
            return [
                pltpu.async_copy(
                    table_hbm.at[ix_ref.at[pl.ds(j * GCH, GCH)]],
                    vl_ref.at[pl.ds(j * GCH, GCH)], sem)
                for j in range(p_cnt // GCH)
            ]

        def acc_into(vl_ref, acc):
            def body(vi, a):
                return a + vl_ref[pl.ds(vi * LANES, LANES)]
            return lax.fori_loop(0, p_cnt // LANES, body, acc)

        ix = [ix_a, ix_b]
        vl = [vl_a, vl_b]
        sems = [sem_a, sem_b]

        acc_p = jnp.zeros((LANES,), jnp.float32)
        acc_n = jnp.zeros((LANES,), jnp.float32)

        compute_idx(pb, 0, ix[0])
        inflight = fire(ix[0], vl[0], sems[0])
        for p in range(1, nph):
            if p == 1:
                for cp in cp_n:
                    cp.wait()
            compute_idx(nb, (p - 1) * p_cnt, ix[p % 2])
            nxt = fire(ix[p % 2], vl[p % 2], sems[p % 2])
            for cp in inflight:
                cp.wait()
            if p - 1 == 0:
                acc_p = acc_into(vl[0], acc_p)
            else:
                acc_n = acc_into(vl[(p - 1) % 2], acc_n)
            inflight = nxt
        for cp in inflight:
            cp.wait()
        acc_n = acc_into(vl[(nph - 1) % 2], acc_n)

        accb[...] = acc_n - float(neg_rate) * acc_p
        pltpu.sync_copy(accb, out_hbm.at[wid])

    return gather_kernel


def kernel(center, pos_word, neg_word, w2v):
    b, s = center.shape
    r = neg_word.shape[1] // s
    v = w2v.shape[0]
    scale = jnp.full((1,), 1.0 / (s * r * b * 2.0), dtype=jnp.float32)
    w_pad = jnp.pad(w2v, ((0, VOCAB_PAD - v), (0, 0)))
    table = _build_table(w_pad, scale).reshape(-1)
    cen_f = center.reshape(-1)
    pos_f = pos_word.reshape(-1)
    negt_f = jnp.swapaxes(neg_word.reshape(b, r, s), 0, 1).reshape(-1)
    gather = _make_gather(cen_f.size, negt_f.size, r)
    partials = gather(table, cen_f, pos_f, negt_f)
    return 1.0 + jnp.sum(partials)


# speedup vs baseline: 1.0253x; 1.0126x over previous
import functools

import jax
import jax.numpy as jnp
from jax import lax
from jax.experimental import pallas as pl
from jax.experimental.pallas import tpu as pltpu
from jax.experimental.pallas import tpu_sc as plsc

VOCAB_PAD = 1024
GCH = 128
LANES = 16


def _table_body(scale_ref, w_ref, out_ref):
    w = w_ref[...]
    nsq = jnp.sum(w * w, axis=1, keepdims=True)
    inv = 1.0 / jnp.maximum(jnp.sqrt(nsq), 1e-8)
    nrm = w * inv
    for i in range(VOCAB_PAD // 128):
        nrmb = nrm[i * 128:(i + 1) * 128, :]
        g = lax.dot_general(nrm, nrmb, (((1,), (1,)), ((), ())),
                            preferred_element_type=jnp.float32,
                            precision=lax.Precision.HIGHEST)
        out_ref[:, i, :] = g * scale_ref[0]


def _build_table(w_pad, scale):
    nblk = VOCAB_PAD // 128
    return pl.pallas_call(
        _table_body,
        out_shape=jax.ShapeDtypeStruct((VOCAB_PAD, nblk, 128), jnp.float32),
        in_specs=[
            pl.BlockSpec(memory_space=pltpu.SMEM),
            pl.BlockSpec(memory_space=pltpu.VMEM),
        ],
        out_specs=pl.BlockSpec(memory_space=pltpu.VMEM),
    )(scale, w_pad)


@functools.lru_cache(maxsize=None)
def _make_gather(num_pos, num_neg, neg_rate):
    info = plsc.get_sparse_core_info()
    nc, ns = info.num_cores, info.num_subcores
    nw = nc * ns
    p_cnt = num_pos // nw
    assert p_cnt * nw == num_pos and num_neg == num_pos * neg_rate
    assert p_cnt % GCH == 0 and p_cnt % LANES == 0
    nph = 1 + neg_rate
    mesh = plsc.VectorSubcoreMesh(core_axis_name="c", subcore_axis_name="s")

    @functools.partial(
        pl.kernel, mesh=mesh,
        out_type=jax.ShapeDtypeStruct((nw, LANES), jnp.float32),
        scratch_types=[
            pltpu.VMEM((p_cnt,), jnp.int32),
            pltpu.VMEM((p_cnt,), jnp.int32),
            pltpu.VMEM((neg_rate * p_cnt,), jnp.int32),
            pltpu.VMEM((p_cnt,), jnp.int32),
            pltpu.VMEM((p_cnt,), jnp.int32),
            pltpu.VMEM((p_cnt,), jnp.float32),
            pltpu.VMEM((p_cnt,), jnp.float32),
            pltpu.VMEM((LANES,), jnp.float32),
            pltpu.SemaphoreType.DMA,
            pltpu.SemaphoreType.DMA,
            pltpu.SemaphoreType.DMA,
            pltpu.SemaphoreType.DMA,
        ],
    )
    def gather_kernel(table_hbm, cen_hbm, pos_hbm, negt_hbm, out_hbm,
                      cb, pb, nb, ix_a, ix_b, vl_a, vl_b, accb,
                      sem_cp, sem_ng, sem_a, sem_b):
        wid = lax.axis_index("s") * nc + lax.axis_index("c")
        base = wid * p_cnt
        cp_c = pltpu.async_copy(cen_hbm.at[pl.ds(base, p_cnt)], cb, sem_cp)
        cp_p = pltpu.async_copy(pos_hbm.at[pl.ds(base, p_cnt)], pb, sem_cp)
        cp_n = [
            pltpu.async_copy(
                negt_hbm.at[pl.ds(rr * num_pos + base, p_cnt)],
                nb.at[pl.ds(rr * p_cnt, p_cnt)], sem_ng)
            for rr in range(neg_rate)
        ]
        cp_c.wait()
        cp_p.wait()

        def compute_idx(oth_ref, oth_off, ix_ref):
            def body(vi, carry):
                st = vi * LANES
                ix_ref[pl.ds(st, LANES)] = (
                    cb[pl.ds(st, LANES)] * VOCAB_PAD
                    + oth_ref[pl.ds(oth_off + st, LANES)])
                return carry
            lax.fori_loop(0, p_cnt // LANES, body, 0)

        def fire(ix_ref, vl_ref, sem):
            return [
                pltpu.async_copy(
                    table_hbm.at[ix_ref.at[pl.ds(j * GCH, GCH)]],
                    vl_ref.at[pl.ds(j * GCH, GCH)], sem)
                for j in range(p_cnt // GCH)
            ]

        def acc_into(vl_ref, acc):
            def body(vi, a):
                return a + vl_ref[pl.ds(vi * LANES, LANES)]
            return lax.fori_loop(0, p_cnt // LANES, body, acc)

        ix = [ix_a, ix_b]
        vl = [vl_a, vl_b]
        sems = [sem_a, sem_b]

        acc_p = jnp.zeros((LANES,), jnp.float32)
        acc_n = jnp.zeros((LANES,), jnp.float32)

        compute_idx(pb, 0, ix[0])
        inflight = fire(ix[0], vl[0], sems[0])
        for p in range(1, nph):
            if p == 1:
                for cp in cp_n:
                    cp.wait()
            compute_idx(nb, (p - 1) * p_cnt, ix[p % 2])
            nxt = fire(ix[p % 2], vl[p % 2], sems[p % 2])
            for cp in inflight:
                cp.wait()
            if p - 1 == 0:
                acc_p = acc_into(vl[0], acc_p)
            else:
                acc_n = acc_into(vl[(p - 1) % 2], acc_n)
            inflight = nxt
        for cp in inflight:
            cp.wait()
        acc_n = acc_into(vl[(nph - 1) % 2], acc_n)

        accb[...] = acc_n - float(neg_rate) * acc_p
        pltpu.sync_copy(accb, out_hbm.at[wid])

    return gather_kernel


def kernel(center, pos_word, neg_word, w2v):
    b, s = center.shape
    r = neg_word.shape[1] // s
    v = w2v.shape[0]
    scale = jnp.full((1,), 1.0 / (s * r * b * 2.0), dtype=jnp.float32)
    w_pad = jnp.pad(w2v, ((0, VOCAB_PAD - v), (0, 0)))
    table = _build_table(w_pad, scale).reshape(-1)
    cen_f = center.reshape(-1)
    pos_f = pos_word.reshape(-1)
    negt_f = jnp.swapaxes(neg_word.reshape(b, r, s), 0, 1).reshape(-1)
    gather = _make_gather(cen_f.size, negt_f.size, r)
    partials = gather(table, cen_f, pos_f, negt_f)
    return 1.0 + jnp.sum(partials)


# speedup vs baseline: 1.1323x; 1.1044x over previous
import functools

import jax
import jax.numpy as jnp
from jax import lax
from jax.experimental import pallas as pl
from jax.experimental.pallas import tpu as pltpu
from jax.experimental.pallas import tpu_sc as plsc

VOCAB_PAD = 1024
GCH = 128
LANES = 16


def _table_body(scale_ref, w_ref, out_ref):
    w = w_ref[...]
    nsq = jnp.sum(w * w, axis=1, keepdims=True)
    inv = 1.0 / jnp.maximum(jnp.sqrt(nsq), 1e-8)
    nrm = w * inv
    for i in range(VOCAB_PAD // 128):
        nrmb = nrm[i * 128:(i + 1) * 128, :]
        g = lax.dot_general(nrm, nrmb, (((1,), (1,)), ((), ())),
                            preferred_element_type=jnp.float32,
                            precision=lax.Precision.HIGHEST)
        out_ref[:, i, :] = g * scale_ref[0]


def _build_table(w_pad, scale):
    nblk = VOCAB_PAD // 128
    return pl.pallas_call(
        _table_body,
        out_shape=jax.ShapeDtypeStruct((VOCAB_PAD, nblk, 128), jnp.float32),
        in_specs=[
            pl.BlockSpec(memory_space=pltpu.SMEM),
            pl.BlockSpec(memory_space=pltpu.VMEM),
        ],
        out_specs=pl.BlockSpec(memory_space=pltpu.VMEM),
    )(scale, w_pad)


@functools.lru_cache(maxsize=None)
def _make_gather(batch, samp, neg_rate):
    info = plsc.get_sparse_core_info()
    nc, ns = info.num_cores, info.num_subcores
    nw = nc * ns
    rows = batch // nw
    p_cnt = rows * samp
    nrow = samp * neg_rate
    assert rows * nw == batch
    assert p_cnt % GCH == 0 and p_cnt % LANES == 0
    assert samp >= 2 * LANES - samp
    nph = 1 + neg_rate
    mesh = plsc.VectorSubcoreMesh(core_axis_name="c", subcore_axis_name="s")

    @functools.partial(
        pl.kernel, mesh=mesh,
        out_type=jax.ShapeDtypeStruct((nw, LANES), jnp.float32),
        compiler_params=pltpu.CompilerParams(use_tc_tiling_on_sc=False),
        scratch_types=[
            pltpu.VMEM((rows, samp), jnp.int32),
            pltpu.VMEM((rows, samp), jnp.int32),
            pltpu.VMEM((rows, nrow), jnp.int32),
            pltpu.VMEM((p_cnt,), jnp.int32),
            pltpu.VMEM((p_cnt,), jnp.int32),
            pltpu.VMEM((p_cnt,), jnp.float32),
            pltpu.VMEM((p_cnt,), jnp.float32),
            pltpu.VMEM((LANES,), jnp.float32),
            pltpu.SemaphoreType.DMA,
            pltpu.SemaphoreType.DMA,
            pltpu.SemaphoreType.DMA,
            pltpu.SemaphoreType.DMA,
        ],
    )
    def gather_kernel(table_hbm, cen_hbm, pos_hbm, neg_hbm, out_hbm,
                      cb, pb, nb, ix_a, ix_b, vl_a, vl_b, accb,
                      sem_cp, sem_ng, sem_a, sem_b):
        wid = lax.axis_index("s") * nc + lax.axis_index("c")
        r0 = wid * rows
        cp_c = pltpu.async_copy(cen_hbm.at[pl.ds(r0, rows), :], cb, sem_cp)
        cp_p = pltpu.async_copy(pos_hbm.at[pl.ds(r0, rows), :], pb, sem_cp)
        cp_n = pltpu.async_copy(neg_hbm.at[pl.ds(r0, rows), :], nb, sem_ng)
        cp_c.wait()
        cp_p.wait()

        hi = samp - LANES

        def compute_idx(oth_ref, col0, ix_ref):
            def body(i, carry):
                c_lo = cb[i, pl.ds(0, LANES)]
                c_hi = cb[i, pl.ds(hi, LANES)]
                o_lo = oth_ref[i, pl.ds(col0, LANES)]
                o_hi = oth_ref[i, pl.ds(col0 + hi, LANES)]
                ix_ref[pl.ds(i * samp + hi, LANES)] = c_hi * VOCAB_PAD + o_hi
                ix_ref[pl.ds(i * samp, LANES)] = c_lo * VOCAB_PAD + o_lo
                return carry
            lax.fori_loop(0, rows, body, 0)

        def fire(ix_ref, vl_ref, sem):
            return [
                pltpu.async_copy(
                    table_hbm.at[ix_ref.at[pl.ds(j * GCH, GCH)]],
                    vl_ref.at[pl.ds(j * GCH, GCH)], sem)
                for j in range(p_cnt // GCH)
            ]

        def acc_into(vl_ref, acc):
            def body(vi, a):
                return a + vl_ref[pl.ds(vi * LANES, LANES)]
            return lax.fori_loop(0, p_cnt // LANES, body, acc)

        ix = [ix_a, ix_b]
        vl = [vl_a, vl_b]
        sems = [sem_a, sem_b]

        acc_p = jnp.zeros((LANES,), jnp.float32)
        acc_n = jnp.zeros((LANES,), jnp.float32)

        compute_idx(pb, 0, ix[0])
        inflight = fire(ix[0], vl[0], sems[0])
        for p in range(1, nph):
            if p == 1:
                cp_n.wait()
            compute_idx(nb, (p - 1) * samp, ix[p % 2])
            nxt = fire(ix[p % 2], vl[p % 2], sems[p % 2])
            for cp in inflight:
                cp.wait()
            if p - 1 == 0:
                acc_p = acc_into(vl[0], acc_p)
            else:
                acc_n = acc_into(vl[(p - 1) % 2], acc_n)
            inflight = nxt
        for cp in inflight:
            cp.wait()
        acc_n = acc_into(vl[(nph - 1) % 2], acc_n)

        accb[...] = acc_n - float(neg_rate) * acc_p
        pltpu.sync_copy(accb, out_hbm.at[wid])

    return gather_kernel


def kernel(center, pos_word, neg_word, w2v):
    b, s = center.shape
    r = neg_word.shape[1] // s
    v = w2v.shape[0]
    scale = jnp.full((1,), 1.0 / (s * r * b * 2.0), dtype=jnp.float32)
    w_pad = jnp.pad(w2v, ((0, VOCAB_PAD - v), (0, 0)))
    table = _build_table(w_pad, scale).reshape(-1)
    gather = _make_gather(b, s, r)
    partials = gather(table, center, pos_word, neg_word)
    return 1.0 + jnp.sum(partials)


# speedup vs baseline: 1.2192x; 1.0767x over previous
import functools

import jax
import jax.numpy as jnp
from jax import lax
from jax.experimental import pallas as pl
from jax.experimental.pallas import tpu as pltpu
from jax.experimental.pallas import tpu_sc as plsc

VOCAB_PAD = 1024
GCH = 128
LANES = 16


def _table_body(scale_ref, w_ref, out_ref):
    w = w_ref[...]
    nsq = jnp.sum(w * w, axis=1, keepdims=True)
    inv = 1.0 / jnp.maximum(jnp.sqrt(nsq), 1e-8)
    nrm = w * inv
    for i in range(VOCAB_PAD // 128):
        nrmb = nrm[i * 128:(i + 1) * 128, :]
        g = lax.dot_general(nrm, nrmb, (((1,), (1,)), ((), ())),
                            preferred_element_type=jnp.float32,
                            precision=lax.Precision.HIGHEST)
        out_ref[:, i, :] = g * scale_ref[0]


def _build_table(w_pad, scale):
    nblk = VOCAB_PAD // 128
    return pl.pallas_call(
        _table_body,
        out_shape=jax.ShapeDtypeStruct((VOCAB_PAD, nblk, 128), jnp.float32),
        in_specs=[
            pl.BlockSpec(memory_space=pltpu.SMEM),
            pl.BlockSpec(memory_space=pltpu.VMEM),
        ],
        out_specs=pl.BlockSpec(memory_space=pltpu.VMEM),
    )(scale, w_pad)


@functools.lru_cache(maxsize=None)
def _make_gather(batch, samp, neg_rate):
    info = plsc.get_sparse_core_info()
    nc, ns = info.num_cores, info.num_subcores
    nw = nc * ns
    rows = batch // nw
    p_cnt = rows * samp
    nrow = samp * neg_rate
    assert rows * nw == batch
    assert p_cnt % GCH == 0 and p_cnt % LANES == 0
    assert samp >= 2 * LANES - samp
    nph = 1 + neg_rate
    mesh = plsc.VectorSubcoreMesh(core_axis_name="c", subcore_axis_name="s")

    @functools.partial(
        pl.kernel, mesh=mesh,
        out_type=jax.ShapeDtypeStruct((nw, LANES), jnp.float32),
        compiler_params=pltpu.CompilerParams(use_tc_tiling_on_sc=False),
        scratch_types=[
            pltpu.VMEM((p_cnt,), jnp.int32),
            pltpu.VMEM((p_cnt,), jnp.int32),
            pltpu.VMEM((rows * nrow,), jnp.int32),
            pltpu.VMEM((p_cnt,), jnp.int32),
            pltpu.VMEM((p_cnt,), jnp.int32),
            pltpu.VMEM((p_cnt,), jnp.float32),
            pltpu.VMEM((p_cnt,), jnp.float32),
            pltpu.VMEM((LANES,), jnp.float32),
            pltpu.SemaphoreType.DMA,
            pltpu.SemaphoreType.DMA,
            pltpu.SemaphoreType.DMA,
            pltpu.SemaphoreType.DMA,
        ],
    )
    def gather_kernel(table_hbm, cen_hbm, pos_hbm, neg_hbm, out_hbm,
                      cb, pb, nb, ix_a, ix_b, vl_a, vl_b, accb,
                      sem_cp, sem_ng, sem_a, sem_b):
        wid = lax.axis_index("s") * nc + lax.axis_index("c")
        r0 = wid * rows
        cp_c = pltpu.async_copy(cen_hbm.at[pl.ds(r0 * samp, p_cnt)], cb,
                                sem_cp)
        cp_p = pltpu.async_copy(pos_hbm.at[pl.ds(r0 * samp, p_cnt)], pb,
                                sem_cp)
        cp_n = pltpu.async_copy(neg_hbm.at[pl.ds(r0 * nrow, rows * nrow)], nb,
                                sem_ng)
        cp_c.wait()
        cp_p.wait()

        hi = samp - LANES

        def compute_idx(oth_ref, ow, col0, ix_ref):
            def body(i, carry):
                c_lo = cb[pl.ds(i * samp, LANES)]
                c_hi = cb[pl.ds(i * samp + hi, LANES)]
                o_lo = oth_ref[pl.ds(i * ow + col0, LANES)]
                o_hi = oth_ref[pl.ds(i * ow + col0 + hi, LANES)]
                ix_ref[pl.ds(i * samp + hi, LANES)] = c_hi * VOCAB_PAD + o_hi
                ix_ref[pl.ds(i * samp, LANES)] = c_lo * VOCAB_PAD + o_lo
                return carry
            lax.fori_loop(0, rows, body, 0)

        def fire(ix_ref, vl_ref, sem):
            return [
                pltpu.async_copy(
                    table_hbm.at[ix_ref.at[pl.ds(j * GCH, GCH)]],
                    vl_ref.at[pl.ds(j * GCH, GCH)], sem)
                for j in range(p_cnt // GCH)
            ]

        def acc_into(vl_ref, acc):
            def body(vi, a):
                return a + vl_ref[pl.ds(vi * LANES, LANES)]
            return lax.fori_loop(0, p_cnt // LANES, body, acc)

        ix = [ix_a, ix_b]
        vl = [vl_a, vl_b]
        sems = [sem_a, sem_b]

        acc_p = jnp.zeros((LANES,), jnp.float32)
        acc_n = jnp.zeros((LANES,), jnp.float32)

        compute_idx(pb, samp, 0, ix[0])
        inflight = fire(ix[0], vl[0], sems[0])
        for p in range(1, nph):
            if p == 1:
                cp_n.wait()
            compute_idx(nb, nrow, (p - 1) * samp, ix[p % 2])
            nxt = fire(ix[p % 2], vl[p % 2], sems[p % 2])
            for cp in inflight:
                cp.wait()
            if p - 1 == 0:
                acc_p = acc_into(vl[0], acc_p)
            else:
                acc_n = acc_into(vl[(p - 1) % 2], acc_n)
            inflight = nxt
        for cp in inflight:
            cp.wait()
        acc_n = acc_into(vl[(nph - 1) % 2], acc_n)

        accb[...] = acc_n - float(neg_rate) * acc_p
        pltpu.sync_copy(accb, out_hbm.at[wid])

    return gather_kernel


def kernel(center, pos_word, neg_word, w2v):
    b, s = center.shape
    r = neg_word.shape[1] // s
    v = w2v.shape[0]
    scale = jnp.full((1,), 1.0 / (s * r * b * 2.0), dtype=jnp.float32)
    w_pad = jnp.pad(w2v, ((0, VOCAB_PAD - v), (0, 0)))
    table = _build_table(w_pad, scale).reshape(-1)
    gather = _make_gather(b, s, r)
    partials = gather(table, center.reshape(-1), pos_word.reshape(-1),
                      neg_word.reshape(-1))
    return 1.0 + jnp.sum(partials)
